# baseline (speedup 1.0000x reference)
import functools

import jax
import jax.numpy as jnp
from jax import lax
from jax.experimental import pallas as pl
from jax.experimental.pallas import tpu as pltpu
from jax.experimental.pallas import tpu_sc as plsc

NC = 2
NS = 16
NW = NC * NS
LANES = 16

BATCH = 32
CHANNELS = 3
PLANE = 512 * 512
CHUNK = 32768
CPP = PLANE // CHUNK
NBINS = 16
FEAT = CHANNELS * NBINS
OUT_DIM = 64
UNROLL = 8


def _sc_body(x_hbm, w_hbm, bias_hbm, out_hbm,
             buf0, buf1, hist_v, w_v, bias_v, out_v,
             sem0, sem1):
  b = lax.axis_index("s") * NC + lax.axis_index("c")

  bufs = (buf0, buf1)
  sems = (sem0, sem1)

  def start_dma(t):
    c, ch = divmod(t, CPP)
    return pltpu.async_copy(
        x_hbm.at[b * CHANNELS + c, pl.ds(ch * CHUNK, CHUNK)],
        bufs[t % 2], sems[t % 2])

  pending = start_dma(0)
  pltpu.sync_copy(w_hbm, w_v)
  pltpu.sync_copy(bias_hbm, bias_v)

  lane = lax.iota(jnp.int32, LANES)
  ones = jnp.ones((LANES,), jnp.float32)
  zeros = jnp.zeros((LANES,), jnp.float32)
  lanes_u = [lane + u * LANES for u in range(UNROLL)]

  feats = []
  for c in range(CHANNELS):
    for l in range(UNROLL * LANES):
      hist_v[l, :] = zeros
    for ch in range(CPP):
      t = c * CPP + ch
      nxt = start_dma(t + 1) if t + 1 < CHANNELS * CPP else None
      pending.wait()
      buf = bufs[t % 2]

      def body(i, _, buf=buf):
        base = i * (LANES * UNROLL)
        for u in range(UNROLL):
          v = buf[pl.ds(base + u * LANES, LANES)]
          bins = (v * 16.0).astype(jnp.int32)
          plsc.addupdate_scatter(hist_v, [lanes_u[u], bins], ones)
        return 0

      lax.fori_loop(0, CHUNK // (LANES * UNROLL), body, 0)
      pending = nxt
    acc = hist_v[0, :]
    for l in range(1, UNROLL * LANES):
      acc = acc + hist_v[l, :]
    feats.append(acc * (1.0 / PLANE))

  for j in range(OUT_DIM // LANES):
    acc = bias_v[pl.ds(j * LANES, LANES)]
    for c in range(CHANNELS):
      for t in range(NBINS):
        k = c * NBINS + t
        acc = acc + feats[c][t] * w_v[k, pl.ds(j * LANES, LANES)]
    out_v[pl.ds(j * LANES, LANES)] = jnp.maximum(acc, 0.0)
  pltpu.sync_copy(out_v, out_hbm.at[b])


@jax.jit
def kernel(x, W, b):
  x2 = x.reshape(BATCH * CHANNELS, PLANE)
  mesh = plsc.VectorSubcoreMesh(core_axis_name="c", subcore_axis_name="s")
  fn = pl.kernel(
      _sc_body,
      out_type=jax.ShapeDtypeStruct((BATCH, OUT_DIM), jnp.float32),
      mesh=mesh,
      compiler_params=pltpu.CompilerParams(needs_layout_passes=False),
      scratch_types=[
          pltpu.VMEM((CHUNK,), jnp.float32),
          pltpu.VMEM((CHUNK,), jnp.float32),
          pltpu.VMEM((UNROLL * LANES, NBINS), jnp.float32),
          pltpu.VMEM((FEAT, OUT_DIM), jnp.float32),
          pltpu.VMEM((OUT_DIM,), jnp.float32),
          pltpu.VMEM((OUT_DIM,), jnp.float32),
          pltpu.SemaphoreType.DMA,
          pltpu.SemaphoreType.DMA,
      ],
  )
  return fn(x2, W, b)


# speedup vs baseline: 2.9570x; 2.9570x over previous
import functools

import jax
import jax.numpy as jnp
from jax import lax
from jax.experimental import pallas as pl
from jax.experimental.pallas import tpu as pltpu
from jax.experimental.pallas import tpu_sc as plsc

NC = 2
NS = 16
NW = NC * NS
LANES = 16

BATCH = 32
CHANNELS = 3
PLANE = 512 * 512
CHUNK = 32768
CPP = PLANE // CHUNK
NBINS = 16
FEAT = CHANNELS * NBINS
OUT_DIM = 64
UNROLL = 8


def _sc_body(x_hbm, w_hbm, bias_hbm, out_hbm,
             buf0, buf1, hist_v, w_v, bias_v, out_v,
             sem0, sem1):
  b = lax.axis_index("s") * NC + lax.axis_index("c")

  bufs = (buf0, buf1)
  sems = (sem0, sem1)

  def start_dma(t):
    c, ch = divmod(t, CPP)
    return pltpu.async_copy(
        x_hbm.at[b * CHANNELS + c, pl.ds(ch * CHUNK, CHUNK)],
        bufs[t % 2], sems[t % 2])

  pending = start_dma(0)
  pltpu.sync_copy(w_hbm, w_v)
  pltpu.sync_copy(bias_hbm, bias_v)

  lane = lax.iota(jnp.int32, LANES)
  ones = jnp.ones((LANES,), jnp.float32)
  zeros = jnp.zeros((LANES,), jnp.float32)
  feats = []
  for c in range(CHANNELS):
    for l in range(LANES):
      hist_v[l, :] = zeros
    for ch in range(CPP):
      t = c * CPP + ch
      nxt = start_dma(t + 1) if t + 1 < CHANNELS * CPP else None
      pending.wait()
      buf = bufs[t % 2]

      @plsc.parallel_loop(0, CHUNK, step=LANES, unroll=UNROLL)
      def _(i, buf=buf):
        v = buf[pl.ds(i, LANES)]
        bins = (v * 16.0).astype(jnp.int32)
        plsc.addupdate_scatter(hist_v, [lane, bins], ones)

      pending = nxt
    acc = hist_v[0, :]
    for l in range(1, LANES):
      acc = acc + hist_v[l, :]
    feats.append(acc * (1.0 / PLANE))

  for j in range(OUT_DIM // LANES):
    acc = bias_v[pl.ds(j * LANES, LANES)]
    for c in range(CHANNELS):
      for t in range(NBINS):
        k = c * NBINS + t
        acc = acc + feats[c][t] * w_v[k, pl.ds(j * LANES, LANES)]
    out_v[pl.ds(j * LANES, LANES)] = jnp.maximum(acc, 0.0)
  pltpu.sync_copy(out_v, out_hbm.at[b])


@jax.jit
def kernel(x, W, b):
  x2 = x.reshape(BATCH * CHANNELS, PLANE)
  mesh = plsc.VectorSubcoreMesh(core_axis_name="c", subcore_axis_name="s")
  fn = pl.kernel(
      _sc_body,
      out_type=jax.ShapeDtypeStruct((BATCH, OUT_DIM), jnp.float32),
      mesh=mesh,
      compiler_params=pltpu.CompilerParams(needs_layout_passes=False),
      scratch_types=[
          pltpu.VMEM((CHUNK,), jnp.float32),
          pltpu.VMEM((CHUNK,), jnp.float32),
          pltpu.VMEM((LANES, NBINS), jnp.float32),
          pltpu.VMEM((FEAT, OUT_DIM), jnp.float32),
          pltpu.VMEM((OUT_DIM,), jnp.float32),
          pltpu.VMEM((OUT_DIM,), jnp.float32),
          pltpu.SemaphoreType.DMA,
          pltpu.SemaphoreType.DMA,
      ],
  )
  return fn(x2, W, b)


# speedup vs baseline: 2.9752x; 1.0062x over previous
import functools

import jax
import jax.numpy as jnp
from jax import lax
from jax.experimental import pallas as pl
from jax.experimental.pallas import tpu as pltpu
from jax.experimental.pallas import tpu_sc as plsc

NC = 2
NS = 16
NW = NC * NS
LANES = 16

BATCH = 32
CHANNELS = 3
PLANE = 512 * 512
CHUNK = 32768
CPP = PLANE // CHUNK
NBINS = 16
FEAT = CHANNELS * NBINS
OUT_DIM = 64
UNROLL = 8


def _sc_body(x_hbm, w_hbm, bias_hbm, out_hbm,
             buf0, buf1, hist_v, w_v, bias_v, out_v,
             sem0, sem1):
  b = lax.axis_index("s") * NC + lax.axis_index("c")

  bufs = (buf0, buf1)
  sems = (sem0, sem1)

  def start_dma(t):
    c, ch = divmod(t, CPP)
    off = (b * CHANNELS + c) * PLANE + ch * CHUNK
    return pltpu.async_copy(
        x_hbm.at[pl.ds(off, CHUNK)], bufs[t % 2], sems[t % 2])

  pending = start_dma(0)
  pltpu.sync_copy(w_hbm, w_v)
  pltpu.sync_copy(bias_hbm, bias_v)

  lane = lax.iota(jnp.int32, LANES)
  ones = jnp.ones((LANES,), jnp.float32)
  zeros = jnp.zeros((LANES,), jnp.float32)
  feats = []
  for c in range(CHANNELS):
    for l in range(LANES):
      hist_v[l, :] = zeros
    for ch in range(CPP):
      t = c * CPP + ch
      nxt = start_dma(t + 1) if t + 1 < CHANNELS * CPP else None
      pending.wait()
      buf = bufs[t % 2]

      @plsc.parallel_loop(0, CHUNK, step=LANES, unroll=UNROLL)
      def _(i, buf=buf):
        v = buf[pl.ds(i, LANES)]
        bins = (v * 16.0).astype(jnp.int32)
        plsc.addupdate_scatter(hist_v, [lane, bins], ones)

      pending = nxt
    acc = hist_v[0, :]
    for l in range(1, LANES):
      acc = acc + hist_v[l, :]
    feats.append(acc * (1.0 / PLANE))

  for j in range(OUT_DIM // LANES):
    acc = bias_v[pl.ds(j * LANES, LANES)]
    for c in range(CHANNELS):
      for t in range(NBINS):
        k = c * NBINS + t
        acc = acc + feats[c][t] * w_v[k, pl.ds(j * LANES, LANES)]
    out_v[pl.ds(j * LANES, LANES)] = jnp.maximum(acc, 0.0)
  pltpu.sync_copy(out_v, out_hbm.at[b])


@jax.jit
def kernel(x, W, b):
  x2 = x.reshape(-1)
  mesh = plsc.VectorSubcoreMesh(core_axis_name="c", subcore_axis_name="s")
  fn = pl.kernel(
      _sc_body,
      out_type=jax.ShapeDtypeStruct((BATCH, OUT_DIM), jnp.float32),
      mesh=mesh,
      compiler_params=pltpu.CompilerParams(needs_layout_passes=False),
      scratch_types=[
          pltpu.VMEM((CHUNK,), jnp.float32),
          pltpu.VMEM((CHUNK,), jnp.float32),
          pltpu.VMEM((LANES, NBINS), jnp.float32),
          pltpu.VMEM((FEAT, OUT_DIM), jnp.float32),
          pltpu.VMEM((OUT_DIM,), jnp.float32),
          pltpu.VMEM((OUT_DIM,), jnp.float32),
          pltpu.SemaphoreType.DMA,
          pltpu.SemaphoreType.DMA,
      ],
  )
  return fn(x2, W, b)


# speedup vs baseline: 4.4314x; 1.4894x over previous
import functools

import jax
import jax.numpy as jnp
from jax import lax
from jax.experimental import pallas as pl
from jax.experimental.pallas import tpu as pltpu
from jax.experimental.pallas import tpu_sc as plsc

NC = 2
NS = 16
NW = NC * NS
LANES = 16

BATCH = 32
CHANNELS = 3
PLANE = 512 * 512
CHUNK = 32768
ROWS = CHUNK // 512
CPP = PLANE // CHUNK
NBINS = 16
FEAT = CHANNELS * NBINS
OUT_DIM = 64
UNROLL = 8


def _sc_body(x_hbm, w_hbm, bias_hbm, out_hbm,
             buf0, buf1, hist_v, w_v, bias_v, out_v,
             sem0, sem1):
  b = lax.axis_index("s") * NC + lax.axis_index("c")

  bufs = (buf0, buf1)
  sems = (sem0, sem1)

  def start_dma(t):
    c, ch = divmod(t, CPP)
    return pltpu.async_copy(
        x_hbm.at[b, c, pl.ds(ch * ROWS, ROWS), :], bufs[t % 2], sems[t % 2])

  pending = start_dma(0)
  pltpu.sync_copy(w_hbm, w_v)
  pltpu.sync_copy(bias_hbm, bias_v)

  lane = lax.iota(jnp.int32, LANES)
  ones = jnp.ones((LANES,), jnp.float32)
  zeros = jnp.zeros((LANES,), jnp.float32)
  feats = []
  for c in range(CHANNELS):
    for l in range(LANES):
      hist_v[l, :] = zeros
    for ch in range(CPP):
      t = c * CPP + ch
      nxt = start_dma(t + 1) if t + 1 < CHANNELS * CPP else None
      pending.wait()
      buf = bufs[t % 2]

      @plsc.parallel_loop(0, CHUNK, step=LANES, unroll=UNROLL)
      def _(i, buf=buf):
        row = lax.shift_right_logical(i, 9)
        col = lax.bitwise_and(i, 511)
        v = buf[row, pl.ds(col, LANES)]
        bins = (v * 16.0).astype(jnp.int32)
        plsc.addupdate_scatter(hist_v, [lane, bins], ones)

      pending = nxt
    acc = hist_v[0, :]
    for l in range(1, LANES):
      acc = acc + hist_v[l, :]
    feats.append(acc * (1.0 / PLANE))

  for j in range(OUT_DIM // LANES):
    acc = bias_v[pl.ds(j * LANES, LANES)]
    for c in range(CHANNELS):
      for t in range(NBINS):
        k = c * NBINS + t
        acc = acc + feats[c][t] * w_v[k, pl.ds(j * LANES, LANES)]
    out_v[pl.ds(j * LANES, LANES)] = jnp.maximum(acc, 0.0)
  pltpu.sync_copy(out_v, out_hbm.at[b])


@jax.jit
def kernel(x, W, b):
  x2 = x
  mesh = plsc.VectorSubcoreMesh(core_axis_name="c", subcore_axis_name="s")
  fn = pl.kernel(
      _sc_body,
      out_type=jax.ShapeDtypeStruct((BATCH, OUT_DIM), jnp.float32),
      mesh=mesh,
      compiler_params=pltpu.CompilerParams(
          needs_layout_passes=False, use_tc_tiling_on_sc=True),
      scratch_types=[
          pltpu.VMEM((ROWS, 512), jnp.float32),
          pltpu.VMEM((ROWS, 512), jnp.float32),
          pltpu.VMEM((LANES, NBINS), jnp.float32),
          pltpu.VMEM((FEAT, OUT_DIM), jnp.float32),
          pltpu.VMEM((OUT_DIM,), jnp.float32),
          pltpu.VMEM((OUT_DIM,), jnp.float32),
          pltpu.SemaphoreType.DMA,
          pltpu.SemaphoreType.DMA,
      ],
  )
  return fn(x2, W, b)
